# Initial kernel scaffold; baseline (speedup 1.0000x reference)
import jax
import jax.numpy as jnp
from jax.experimental import pallas as pl


def kernel(x, memory, pos_bias):
    raise NotImplementedError("write your pallas kernel here")


# speedup vs baseline: 1.2011x; 1.2011x over previous
import functools

import jax
import jax.numpy as jnp
from jax import lax
from jax.experimental import pallas as pl
from jax.experimental.pallas import tpu as pltpu
from jax.experimental.pallas import tpu_sc as plsc

B, L, D = 4, 2048, 1024
S, N = 512, 3
BL = B * L

T = 256
G = 4
ZROW = S * N

NC, NS = 2, 16
NW = NC * NS
RPW = BL // NW
CH = 16
NCH = RPW // CH


def _tc_argmax_body(xprev_ref, xcur_ref, mem_ref, pbT_ref, out_ref):
    i = pl.program_id(0)
    pos0 = (i * T) % L
    prev = xprev_ref[...]
    cur = xcur_ref[...]
    cols = []
    for n in range(N):
        sh = (N - 1) - n
        if sh:
            xs = jnp.concatenate([prev[T - sh:], cur[: T - sh]], axis=0)
        else:
            xs = cur
        m_n = mem_ref[:, n, :]
        sims = lax.dot_general(
            xs.astype(jnp.bfloat16), m_n.astype(jnp.bfloat16),
            (((1,), (1,)), ((), ())),
            preferred_element_type=jnp.float32)
        rowid = lax.broadcasted_iota(jnp.int32, (T, S), 0)
        sims = jnp.where(rowid + pos0 >= sh, sims, 0.0)
        scores = sims + pbT_ref[n][None, :]
        amax = jnp.max(scores, axis=1, keepdims=True)
        colid = lax.broadcasted_iota(jnp.int32, (T, S), 1)
        best = jnp.min(jnp.where(scores == amax, colid, S),
                       axis=1, keepdims=True)
        cols.append(best * N + n)
    cols.append(jnp.full((T, 1), ZROW, jnp.int32))
    out_ref[...] = jnp.concatenate(cols, axis=1)


def _tc_argmax(xf, memory, pbT):
    return pl.pallas_call(
        _tc_argmax_body,
        grid=(BL // T,),
        in_specs=[
            pl.BlockSpec((T, D), lambda i: (jnp.maximum(i - 1, 0), 0)),
            pl.BlockSpec((T, D), lambda i: (i, 0)),
            pl.BlockSpec((S, N, D), lambda i: (0, 0, 0)),
            pl.BlockSpec((8, S), lambda i: (0, 0)),
        ],
        out_specs=pl.BlockSpec((T, G), lambda i: (i, 0)),
        out_shape=jax.ShapeDtypeStruct((BL, G), jnp.int32),
    )(xf, xf, memory, pbT)


def _sc_gather_body(idx_hbm, mem_hbm, out_hbm, idx_v, g_v, out_v, sem):
    wid = lax.axis_index("s") * NC + lax.axis_index("c")
    base = wid * RPW

    def chunk(ci, carry):
        start = base + ci * CH
        pltpu.sync_copy(idx_hbm.at[pl.ds(start * G, G * CH)], idx_v)
        pltpu.async_copy(mem_hbm.at[idx_v], g_v, sem).wait()

        def row(r, c2):
            rr = G * r
            for j in range(D // 16):
                slc = pl.ds(j * 16, 16)
                out_v[r, slc] = g_v[rr, slc] + g_v[rr + 1, slc] + g_v[rr + 2, slc]
            return c2

        lax.fori_loop(0, CH, row, 0)
        pltpu.sync_copy(out_v, out_hbm.at[pl.ds(start, CH)])
        return carry

    lax.fori_loop(0, NCH, chunk, 0)


def _sc_gather(idx_flat, mem_flat):
    mesh = plsc.VectorSubcoreMesh(core_axis_name="c", subcore_axis_name="s")
    fn = functools.partial(
        pl.kernel,
        mesh=mesh,
        out_type=jax.ShapeDtypeStruct((BL, D), jnp.float32),
        scratch_types=[
            pltpu.VMEM((G * CH,), jnp.int32),
            pltpu.VMEM((G * CH, D), jnp.float32),
            pltpu.VMEM((CH, D), jnp.float32),
            pltpu.SemaphoreType.DMA,
        ],
    )(_sc_gather_body)
    return fn(idx_flat, mem_flat)


def kernel(x, memory, pos_bias):
    xf = x.reshape(BL, D)
    pbT = jnp.pad(pos_bias.T, ((0, 8 - N), (0, 0)))
    idx = _tc_argmax(xf, memory, pbT)
    mem_flat = jnp.concatenate(
        [memory.reshape(S * N, D), jnp.zeros((8, D), jnp.float32)], axis=0)
    out = _sc_gather(idx.reshape(-1), mem_flat)
    return out.reshape(B, L, D)


# speedup vs baseline: 2.3896x; 1.9896x over previous
import functools

import jax
import jax.numpy as jnp
from jax import lax
from jax.experimental import pallas as pl
from jax.experimental.pallas import tpu as pltpu
from jax.experimental.pallas import tpu_sc as plsc

B, L, D = 4, 2048, 1024
S, N = 512, 3
BL = B * L

T = 256

NC, NS = 2, 16
NW = NC * NS
RPW = BL // NW
CH = 64
NCH = RPW // CH


def _tc_argmax_body(xprev_ref, xcur_ref, mem_ref, pbP_ref, out_ref):
    i = pl.program_id(0)
    pos0 = (i * T) % L
    prev = xprev_ref[...]
    cur = xcur_ref[...]
    rows = []
    for n in range(N):
        sh = (N - 1) - n
        if sh:
            xs = jnp.concatenate([prev[T - sh:], cur[: T - sh]], axis=0)
        else:
            xs = cur
        m_n = mem_ref[:, n, :]
        simsT = lax.dot_general(
            m_n.astype(jnp.bfloat16), xs.astype(jnp.bfloat16),
            (((1,), (1,)), ((), ())),
            preferred_element_type=jnp.float32)
        colid = lax.broadcasted_iota(jnp.int32, (S, T), 1)
        simsT = jnp.where(colid + pos0 >= sh, simsT, 0.0)
        scores = simsT + pbP_ref[:, n:n + 1]
        amax = jnp.max(scores, axis=0, keepdims=True)
        rowid = lax.broadcasted_iota(jnp.int32, (S, T), 0)
        best = jnp.min(jnp.where(scores == amax, rowid, S),
                       axis=0, keepdims=True)
        rows.append(best * N + n)
    rows.append(jnp.zeros((8 - N, T), jnp.int32))
    out_ref[...] = jnp.concatenate(rows, axis=0)


def _tc_argmax(xf, memory, pbP):
    return pl.pallas_call(
        _tc_argmax_body,
        grid=(BL // T,),
        in_specs=[
            pl.BlockSpec((T, D), lambda i: (jnp.maximum(i - 1, 0), 0)),
            pl.BlockSpec((T, D), lambda i: (i, 0)),
            pl.BlockSpec((S, N, D), lambda i: (0, 0, 0)),
            pl.BlockSpec((S, 8), lambda i: (0, 0)),
        ],
        out_specs=pl.BlockSpec((8, T), lambda i: (0, i)),
        out_shape=jax.ShapeDtypeStruct((8, BL), jnp.int32),
    )(xf, xf, memory, pbP)


def _sc_gather_body(idx_hbm, mem_hbm, out_hbm, ia0, ia1, ia2, acc, sem):
    wid = lax.axis_index("s") * NC + lax.axis_index("c")
    base = wid * RPW
    for ci in range(NCH):
        s0 = ci * CH
        pltpu.sync_copy(idx_hbm.at[pl.ds(0 * BL + base + s0, CH)], ia0)
        pltpu.sync_copy(idx_hbm.at[pl.ds(1 * BL + base + s0, CH)], ia1)
        pltpu.sync_copy(idx_hbm.at[pl.ds(2 * BL + base + s0, CH)], ia2)
        pltpu.async_copy(mem_hbm.at[ia0], acc, sem).wait()
        pltpu.async_copy(mem_hbm.at[ia1], acc, sem, add=True).wait()
        pltpu.async_copy(mem_hbm.at[ia2], acc, sem, add=True).wait()
        pltpu.sync_copy(acc, out_hbm.at[pl.ds(base + s0, CH)])


def _sc_gather(idx, mem_flat):
    mesh = plsc.VectorSubcoreMesh(core_axis_name="c", subcore_axis_name="s")
    fn = functools.partial(
        pl.kernel,
        mesh=mesh,
        out_type=jax.ShapeDtypeStruct((BL, D // 128, 128), jnp.float32),
        scratch_types=[
            pltpu.VMEM((CH,), jnp.int32),
            pltpu.VMEM((CH,), jnp.int32),
            pltpu.VMEM((CH,), jnp.int32),
            pltpu.VMEM((CH, D // 128, 128), jnp.float32),
            pltpu.SemaphoreType.DMA,
        ],
    )(_sc_gather_body)
    return fn(idx.reshape(-1), mem_flat.reshape(S * N, D // 128, 128))


def kernel(x, memory, pos_bias):
    xf = x.reshape(BL, D)
    pbP = jnp.pad(pos_bias, ((0, 0), (0, 8 - N)))
    idx = _tc_argmax(xf, memory, pbP)
    mem_flat = memory.reshape(S * N, D)
    out = _sc_gather(idx, mem_flat)
    return out.reshape(B, L, D)


# speedup vs baseline: 4.2706x; 1.7871x over previous
import functools

import jax
import jax.numpy as jnp
from jax import lax
from jax.experimental import pallas as pl
from jax.experimental.pallas import tpu as pltpu
from jax.experimental.pallas import tpu_sc as plsc

B, L, D = 4, 2048, 1024
S, N = 512, 3
BL = B * L

T = 256

NC, NS = 2, 16
NW = NC * NS
RPW = BL // NW
CH = 64
NCH = RPW // CH


def _tc_argmax_body(xprev_ref, xcur_ref, mem_ref, pbP_ref, out_ref):
    i = pl.program_id(0)
    pos0 = (i * T) % L
    prev = xprev_ref[...]
    cur = xcur_ref[...]
    rows = []
    for n in range(N):
        sh = (N - 1) - n
        if sh:
            xs = jnp.concatenate([prev[T - sh:], cur[: T - sh]], axis=0)
        else:
            xs = cur
        m_n = mem_ref[n]
        simsT = lax.dot_general(
            m_n, xs, (((1,), (1,)), ((), ())),
            preferred_element_type=jnp.float32)
        colid = lax.broadcasted_iota(jnp.int32, (S, T), 1)
        simsT = jnp.where(colid + pos0 >= sh, simsT, 0.0)
        scores = simsT + pbP_ref[:, n:n + 1]
        amax = jnp.max(scores, axis=0, keepdims=True)
        rowid = lax.broadcasted_iota(jnp.int32, (S, T), 0)
        best = jnp.min(jnp.where(scores == amax, rowid, S),
                       axis=0, keepdims=True)
        rows.append(best * N + n)
    rows.append(jnp.zeros((8 - N, T), jnp.int32))
    out_ref[...] = jnp.concatenate(rows, axis=0)


def _tc_argmax(xf, memory, pbP):
    return pl.pallas_call(
        _tc_argmax_body,
        grid=(BL // T,),
        in_specs=[
            pl.BlockSpec((T, D), lambda i: (jnp.maximum(i - 1, 0), 0)),
            pl.BlockSpec((T, D), lambda i: (i, 0)),
            pl.BlockSpec((N, S, D), lambda i: (0, 0, 0)),
            pl.BlockSpec((S, 8), lambda i: (0, 0)),
        ],
        out_specs=pl.BlockSpec((8, T), lambda i: (0, i)),
        out_shape=jax.ShapeDtypeStruct((8, BL), jnp.int32),
    )(xf, xf, memory, pbP)


def _sc_gather_body(idx_hbm, mem_hbm, out_hbm, ia0, ia1, ia2, acc, sem):
    wid = lax.axis_index("s") * NC + lax.axis_index("c")
    base = wid * RPW
    for ci in range(NCH):
        s0 = ci * CH
        pltpu.sync_copy(idx_hbm.at[pl.ds(0 * BL + base + s0, CH)], ia0)
        pltpu.sync_copy(idx_hbm.at[pl.ds(1 * BL + base + s0, CH)], ia1)
        pltpu.sync_copy(idx_hbm.at[pl.ds(2 * BL + base + s0, CH)], ia2)
        pltpu.async_copy(mem_hbm.at[ia0], acc, sem).wait()
        pltpu.async_copy(mem_hbm.at[ia1], acc, sem, add=True).wait()
        pltpu.async_copy(mem_hbm.at[ia2], acc, sem, add=True).wait()
        pltpu.sync_copy(acc, out_hbm.at[pl.ds(base + s0, CH)])


def _sc_gather(idx, mem_flat):
    mesh = plsc.VectorSubcoreMesh(core_axis_name="c", subcore_axis_name="s")
    fn = functools.partial(
        pl.kernel,
        mesh=mesh,
        out_type=jax.ShapeDtypeStruct((BL, D // 128, 128), jnp.float32),
        scratch_types=[
            pltpu.VMEM((CH,), jnp.int32),
            pltpu.VMEM((CH,), jnp.int32),
            pltpu.VMEM((CH,), jnp.int32),
            pltpu.VMEM((CH, D // 128, 128), jnp.float32),
            pltpu.SemaphoreType.DMA,
        ],
    )(_sc_gather_body)
    return fn(idx.reshape(-1), mem_flat.reshape(S * N, D // 128, 128))


def kernel(x, memory, pos_bias):
    xb = x.reshape(BL, D).astype(jnp.bfloat16)
    memT = jnp.transpose(memory, (1, 0, 2)).astype(jnp.bfloat16)
    pbP = jnp.pad(pos_bias, ((0, 0), (0, 8 - N)))
    idx = _tc_argmax(xb, memT, pbP)
    mem_flat = memory.reshape(S * N, D)
    out = _sc_gather(idx, mem_flat)
    return out.reshape(B, L, D)


# speedup vs baseline: 4.8657x; 1.1394x over previous
import functools

import jax
import jax.numpy as jnp
from jax import lax
from jax.experimental import pallas as pl
from jax.experimental.pallas import tpu as pltpu
from jax.experimental.pallas import tpu_sc as plsc

B, L, D = 4, 2048, 1024
S, N = 512, 3
BL = B * L

T = 256

NC, NS = 2, 16
NW = NC * NS
RPW = BL // NW
CH = 32
NCH = RPW // CH


def _tc_argmax_body(xprev_ref, xcur_ref, mem_ref, pbP_ref, out_ref):
    i = pl.program_id(0)
    pos0 = (i * T) % L
    prev = xprev_ref[...]
    cur = xcur_ref[...]
    rows = []
    for n in range(N):
        sh = (N - 1) - n
        if sh:
            xs = jnp.concatenate([prev[T - sh:], cur[: T - sh]], axis=0)
        else:
            xs = cur
        m_n = mem_ref[n]
        simsT = lax.dot_general(
            m_n, xs.astype(jnp.bfloat16), (((1,), (1,)), ((), ())),
            preferred_element_type=jnp.float32)
        colid = lax.broadcasted_iota(jnp.int32, (S, T), 1)
        simsT = jnp.where(colid + pos0 >= sh, simsT, 0.0)
        scores = simsT + pbP_ref[:, n:n + 1]
        amax = jnp.max(scores, axis=0, keepdims=True)
        rowid = lax.broadcasted_iota(jnp.int32, (S, T), 0)
        best = jnp.min(jnp.where(scores == amax, rowid, S),
                       axis=0, keepdims=True)
        rows.append(best * N + n)
    rows.append(jnp.zeros((8 - N, T), jnp.int32))
    out_ref[...] = jnp.concatenate(rows, axis=0)


def _tc_argmax(xf, memory, pbP):
    return pl.pallas_call(
        _tc_argmax_body,
        grid=(BL // T,),
        in_specs=[
            pl.BlockSpec((T, D), lambda i: (jnp.maximum(i - 1, 0), 0)),
            pl.BlockSpec((T, D), lambda i: (i, 0)),
            pl.BlockSpec((N, S, D), lambda i: (0, 0, 0)),
            pl.BlockSpec((S, 8), lambda i: (0, 0)),
        ],
        out_specs=pl.BlockSpec((8, T), lambda i: (0, i)),
        out_shape=jax.ShapeDtypeStruct((8, BL), jnp.int32),
    )(xf, xf, memory, pbP)


def _sc_gather_body(idx_hbm, mem_hbm, out_hbm,
                    ia0, ia1, ia2, acc0, acc1,
                    sem_g, sem_a, sem_s0, sem_s1):
    wid = lax.axis_index("s") * NC + lax.axis_index("c")
    base = wid * RPW
    accs = (acc0, acc1)
    ssems = (sem_s0, sem_s1)
    pltpu.sync_copy(idx_hbm.at[pl.ds(0 * BL + base, RPW)], ia0)
    pltpu.sync_copy(idx_hbm.at[pl.ds(1 * BL + base, RPW)], ia1)
    pltpu.sync_copy(idx_hbm.at[pl.ds(2 * BL + base, RPW)], ia2)
    g0 = pltpu.async_copy(mem_hbm.at[ia0.at[pl.ds(0, CH)]], acc0, sem_g)
    stores = [None, None]
    for ci in range(NCH):
        b = ci % 2
        acc = accs[b]
        g0.wait()
        a1 = pltpu.async_copy(mem_hbm.at[ia1.at[pl.ds(ci * CH, CH)]],
                              acc, sem_a, add=True)
        a2 = pltpu.async_copy(mem_hbm.at[ia2.at[pl.ds(ci * CH, CH)]],
                              acc, sem_a, add=True)
        if ci + 1 < NCH:
            nb = (ci + 1) % 2
            if stores[nb] is not None:
                stores[nb].wait()
            g0 = pltpu.async_copy(mem_hbm.at[ia0.at[pl.ds((ci + 1) * CH, CH)]],
                                  accs[nb], sem_g)
        a1.wait()
        a2.wait()
        stores[b] = pltpu.make_async_copy(
            acc, out_hbm.at[pl.ds(base + ci * CH, CH)], ssems[b])
        stores[b].start()
    for st in stores:
        if st is not None:
            st.wait()


def _sc_gather(idx, mem_flat):
    mesh = plsc.VectorSubcoreMesh(core_axis_name="c", subcore_axis_name="s")
    fn = functools.partial(
        pl.kernel,
        mesh=mesh,
        out_type=jax.ShapeDtypeStruct((BL, D // 128, 128), jnp.float32),
        scratch_types=[
            pltpu.VMEM((RPW,), jnp.int32),
            pltpu.VMEM((RPW,), jnp.int32),
            pltpu.VMEM((RPW,), jnp.int32),
            pltpu.VMEM((CH, D // 128, 128), jnp.float32),
            pltpu.VMEM((CH, D // 128, 128), jnp.float32),
            pltpu.SemaphoreType.DMA,
            pltpu.SemaphoreType.DMA,
            pltpu.SemaphoreType.DMA,
            pltpu.SemaphoreType.DMA,
        ],
    )(_sc_gather_body)
    return fn(idx.reshape(-1), mem_flat.reshape(S * N, D // 128, 128))


def kernel(x, memory, pos_bias):
    xf = x.reshape(BL, D)
    memT = jnp.transpose(memory, (1, 0, 2)).astype(jnp.bfloat16)
    pbP = jnp.pad(pos_bias, ((0, 0), (0, 8 - N)))
    idx = _tc_argmax(xf, memT, pbP)
    mem_flat = memory.reshape(S * N, D)
    out = _sc_gather(idx, mem_flat)
    return out.reshape(B, L, D)
